# baseline (speedup 1.0000x reference)
import functools

import jax
import jax.numpy as jnp
from jax import lax
from jax.experimental import pallas as pl
from jax.experimental.pallas import tpu as pltpu
from jax.experimental.pallas import tpu_sc as plsc

_VOCAB = 1000
_VPAD = 1024
_EMB = 8
_BATCH = 1024
_SEQ = 50
_SEQ_PAD = 56
_HALF_A = 24
_HALF_B = _SEQ - _HALF_A

_NC = 2
_NS = 16
_NW = _NC * _NS
_NSPLIT = 1
_CHUNK_B = _BATCH // _NSPLIT


def _table_body(embed_ref, w_ref, b_ref, out_ref):
    t = lax.dot_general(
        embed_ref[...], w_ref[...],
        (((1,), (1,)), ((), ())),
        preferred_element_type=jnp.float32,
    ) + b_ref[...]
    out_ref[...] = t.reshape(_VOCAB, 8, 128)


def _make_table3(embed, w_pad, b_pad):
    return pl.pallas_call(
        _table_body,
        out_shape=jax.ShapeDtypeStruct((_VOCAB, 8, 128), jnp.float32),
    )(embed, w_pad, b_pad.reshape(1, _VPAD))


_sc_mesh = plsc.VectorSubcoreMesh(core_axis_name="c", subcore_axis_name="s")

_BATCH_PER_W = _CHUNK_B // _NW
_IDS_PER_W = _BATCH_PER_W * _SEQ_PAD

_SC_SCRATCH = [
    pltpu.VMEM((_IDS_PER_W,), jnp.int32),
    pltpu.VMEM((_HALF_A, 8, 128), jnp.float32),
    pltpu.VMEM((_HALF_B, 8, 128), jnp.float32),
    pltpu.VMEM((_SEQ, _VOCAB), jnp.float32),
    pltpu.SemaphoreType.DMA,
    pltpu.SemaphoreType.DMA,
    pltpu.SemaphoreType.DMA,
]


def _sc_gather_body(t3_hbm, ids_hbm, out_hbm,
                    idx_v, b3a, b3b, buf, sa, sb, sw):
    wid = lax.axis_index("s") * _NC + lax.axis_index("c")
    batch0 = wid * _BATCH_PER_W
    pltpu.sync_copy(ids_hbm.at[pl.ds(wid * _IDS_PER_W, _IDS_PER_W)], idx_v)

    def idx_at(g, local_off, n):
        off = pl.multiple_of(g * _SEQ_PAD + local_off, 8)
        return idx_v.at[pl.ds(off, n)]

    def start(g):
        pltpu.async_copy(t3_hbm.at[idx_at(g, 0, _HALF_A)], b3a, sa)
        pltpu.async_copy(t3_hbm.at[idx_at(g, _HALF_A, _HALF_B)], b3b, sb)

    def wait_half(b3, sem):
        pltpu.make_async_copy(t3_hbm.at[idx_at(0, 0, b3.shape[0])], b3, sem).wait()

    def vec_half(b3, row0, nrows):
        @plsc.parallel_loop(0, nrows, 1, unroll=2)
        def _(rr):
            r = row0 + rr
            for i in range(7):
                vals = [b3[rr, i, pl.ds(16 * j, 16)] for j in range(8)]
                for j in range(8):
                    buf[r, pl.ds(128 * i + 16 * j, 16)] = vals[j]
            vtail = b3[rr, 7, pl.ds(88, 16)]
            vals = [b3[rr, 7, pl.ds(16 * j, 16)] for j in range(6)]
            buf[r, pl.ds(_VOCAB - 16, 16)] = vtail
            for j in range(6):
                buf[r, pl.ds(896 + 16 * j, 16)] = vals[j]

    def wait_write():
        pltpu.make_async_copy(buf, out_hbm.at[batch0], sw).wait()

    start(0)

    def body(g, carry):
        wait_half(b3a, sa)

        @pl.when(g > 0)
        def _():
            wait_write()

        vec_half(b3a, 0, _HALF_A)
        wait_half(b3b, sb)
        vec_half(b3b, _HALF_A, _HALF_B)
        pltpu.async_copy(buf, out_hbm.at[batch0 + g], sw)

        @pl.when(g + 1 < _BATCH_PER_W)
        def _():
            start(g + 1)

        return carry

    lax.fori_loop(0, _BATCH_PER_W, body, 0)
    wait_write()


_sc_gather = pl.kernel(
    _sc_gather_body,
    out_type=jax.ShapeDtypeStruct((_CHUNK_B, _SEQ, _VOCAB), jnp.float32),
    mesh=_sc_mesh,
    scratch_types=_SC_SCRATCH,
)


def kernel(input_ids, embed, W, b):
    w_pad = jnp.pad(W, ((0, _VPAD - _VOCAB), (0, 0)))
    b_pad = jnp.pad(b, (0, _VPAD - _VOCAB))
    t3 = _make_table3(embed, w_pad, b_pad)
    ids_pad = jnp.pad(
        input_ids.astype(jnp.int32), ((0, 0), (0, _SEQ_PAD - _SEQ))
    ).reshape(_BATCH * _SEQ_PAD)
    return _sc_gather(t3, ids_pad)


# speedup vs baseline: 1.0662x; 1.0662x over previous
import functools

import jax
import jax.numpy as jnp
from jax import lax
from jax.experimental import pallas as pl
from jax.experimental.pallas import tpu as pltpu
from jax.experimental.pallas import tpu_sc as plsc

_VOCAB = 1000
_VPAD = 1024
_EMB = 8
_BATCH = 1024
_SEQ = 50
_SEQ_PAD = 56
_HALF_A = 24
_HALF_B = _SEQ - _HALF_A

_NC = 2
_NS = 16
_NW = _NC * _NS
_NSPLIT = 1
_CHUNK_B = _BATCH // _NSPLIT


def _table_body(embed_ref, w_ref, b_ref, out_ref):
    t = lax.dot_general(
        embed_ref[...], w_ref[...],
        (((1,), (1,)), ((), ())),
        preferred_element_type=jnp.float32,
    ) + b_ref[...]
    out_ref[...] = t.reshape(_VOCAB, 8, 128)


def _make_table3(embed, w_pad, b_pad):
    return pl.pallas_call(
        _table_body,
        out_shape=jax.ShapeDtypeStruct((_VOCAB, 8, 128), jnp.float32),
    )(embed, w_pad, b_pad.reshape(1, _VPAD))


_sc_mesh = plsc.VectorSubcoreMesh(core_axis_name="c", subcore_axis_name="s")

_BATCH_PER_W = _CHUNK_B // _NW
_IDS_PER_W = _BATCH_PER_W * _SEQ_PAD

_SC_SCRATCH = [
    pltpu.VMEM((_IDS_PER_W,), jnp.int32),
    pltpu.VMEM((_HALF_A, 8, 128), jnp.float32),
    pltpu.VMEM((_HALF_B, 8, 128), jnp.float32),
    pltpu.VMEM((_SEQ, _VOCAB), jnp.float32),
    pltpu.SemaphoreType.DMA,
    pltpu.SemaphoreType.DMA,
    pltpu.SemaphoreType.DMA,
]


def _sc_gather_body(t3_hbm, ids_hbm, out_hbm,
                    idx_v, b3a, b3b, buf, sa, sb, sw):
    wid = lax.axis_index("s") * _NC + lax.axis_index("c")
    batch0 = wid * _BATCH_PER_W
    pltpu.sync_copy(ids_hbm.at[pl.ds(wid * _IDS_PER_W, _IDS_PER_W)], idx_v)

    def idx_at(g, local_off, n):
        off = pl.multiple_of(g * _SEQ_PAD + local_off, 8)
        return idx_v.at[pl.ds(off, n)]

    def start_a(g):
        pltpu.async_copy(t3_hbm.at[idx_at(g, 0, _HALF_A)], b3a, sa)

    def start_b(g):
        pltpu.async_copy(t3_hbm.at[idx_at(g, _HALF_A, _HALF_B)], b3b, sb)

    def wait_half(b3, sem):
        pltpu.make_async_copy(t3_hbm.at[idx_at(0, 0, b3.shape[0])], b3, sem).wait()

    def vec_half(b3, row0, nrows):
        @plsc.parallel_loop(0, nrows, 1, unroll=2)
        def _(rr):
            r = row0 + rr
            for i in range(7):
                vals = [b3[rr, i, pl.ds(16 * j, 16)] for j in range(8)]
                for j in range(8):
                    buf[r, pl.ds(128 * i + 16 * j, 16)] = vals[j]
            vtail = b3[rr, 7, pl.ds(88, 16)]
            vals = [b3[rr, 7, pl.ds(16 * j, 16)] for j in range(6)]
            buf[r, pl.ds(_VOCAB - 16, 16)] = vtail
            for j in range(6):
                buf[r, pl.ds(896 + 16 * j, 16)] = vals[j]

    def wait_write():
        pltpu.make_async_copy(buf, out_hbm.at[batch0], sw).wait()

    start_a(0)
    start_b(0)

    def body(g, carry):
        wait_half(b3a, sa)

        @pl.when(g > 0)
        def _():
            wait_write()

        vec_half(b3a, 0, _HALF_A)

        @pl.when(g + 1 < _BATCH_PER_W)
        def _():
            start_a(g + 1)

        wait_half(b3b, sb)
        vec_half(b3b, _HALF_A, _HALF_B)

        @pl.when(g + 1 < _BATCH_PER_W)
        def _():
            start_b(g + 1)

        pltpu.async_copy(buf, out_hbm.at[batch0 + g], sw)
        return carry

    lax.fori_loop(0, _BATCH_PER_W, body, 0)
    wait_write()


_sc_gather = pl.kernel(
    _sc_gather_body,
    out_type=jax.ShapeDtypeStruct((_CHUNK_B, _SEQ, _VOCAB), jnp.float32),
    mesh=_sc_mesh,
    scratch_types=_SC_SCRATCH,
)


def kernel(input_ids, embed, W, b):
    w_pad = jnp.pad(W, ((0, _VPAD - _VOCAB), (0, 0)))
    b_pad = jnp.pad(b, (0, _VPAD - _VOCAB))
    t3 = _make_table3(embed, w_pad, b_pad)
    ids_pad = jnp.pad(
        input_ids.astype(jnp.int32), ((0, 0), (0, _SEQ_PAD - _SEQ))
    ).reshape(_BATCH * _SEQ_PAD)
    return _sc_gather(t3, ids_pad)
